# baseline (speedup 1.0000x reference)
import functools

import jax
import jax.numpy as jnp
from jax import lax
from jax.experimental import pallas as pl
from jax.experimental.pallas import tpu as pltpu
from jax.experimental.pallas import tpu_sc as plsc

VOCAB = 100000
EMBED_DIM = 64
NEG_RATIO = 5
BATCH = 1024

NUM_WORKERS = 32
ROWS_W = BATCH // NUM_WORKERS
NEG_W = ROWS_W * NEG_RATIO
ROUNDS = 8
CAND_W = ROUNDS * NEG_W
LANES = 16
CHUNK = 128

_INV24 = 1.0 / 16777216.0


def _fmix32(x):
    x = x ^ (x >> jnp.uint32(16))
    x = x * jnp.uint32(0x85EBCA6B)
    x = x ^ (x >> jnp.uint32(13))
    x = x * jnp.uint32(0xC2B2AE35)
    x = x ^ (x >> jnp.uint32(16))
    return x


def _u01(bits):
    return (bits >> jnp.uint32(8)).astype(jnp.int32).astype(jnp.float32) \
        * jnp.float32(_INV24)


def _tc_pack_emb_body(emb_ref, tgt_ref, out_ref, out_tgt_ref):
    e = emb_ref[...].reshape(BATCH, EMBED_DIM)
    out_ref[...] = jnp.concatenate([e, e], axis=1)
    out_tgt_ref[...] = tgt_ref[...].reshape(8, 128)


def _sc_body(emb_hbm, wf_hbm, tgt_hbm, fc_hbm, posl_out, negl_out,
             tgt2_v, tgt_v, emb_v, jbuf, ubuf, wfbuf, negidx, negrows, posrows,
             poslog, neglog, sem, sem2):
    wid = lax.axis_index("s") * 2 + lax.axis_index("c")
    iota = lax.iota(jnp.int32, LANES)

    pltpu.sync_copy(
        tgt_hbm.at[pl.ds(wid // 4, 1), pl.ds((wid % 4) * ROWS_W, ROWS_W)],
        tgt2_v)
    for k in range(ROWS_W // LANES):
        tgt_v[pl.ds(k * LANES, LANES)] = tgt2_v[0, pl.ds(k * LANES, LANES)]
    pltpu.sync_copy(emb_hbm.at[pl.ds(wid * ROWS_W, ROWS_W)], emb_v)
    cp_pos = pltpu.async_copy(fc_hbm.at[tgt_v], posrows, sem2)

    base_u = (wid * CAND_W).astype(jnp.uint32)
    cps = []
    for ch in range(CAND_W // CHUNK):
        for cv in range(CHUNK // LANES):
            off = ch * CHUNK + cv * LANES
            g = (iota + off).astype(jnp.uint32) + base_u
            jbits = _fmix32(g * jnp.uint32(0x9E3779B1))
            ubits = _fmix32((g ^ jnp.uint32(0xDEADBEEF)) * jnp.uint32(0x85EBCA77))
            j = (_u01(jbits) * jnp.float32(VOCAB)).astype(jnp.int32)
            jbuf[pl.ds(off, LANES)] = j
            ubuf[pl.ds(off, LANES)] = _u01(ubits)
        cps.append(pltpu.async_copy(
            wf_hbm.at[jbuf.at[pl.ds(ch * CHUNK, CHUNK)]],
            wfbuf.at[pl.ds(ch * CHUNK, CHUNK)], sem))
    for cp in cps:
        cp.wait()

    for sv in range(NEG_W // LANES):
        slot = iota + jnp.int32(sv * LANES)
        row = (slot.astype(jnp.float32) * jnp.float32(1.0 / NEG_RATIO)
               ).astype(jnp.int32)
        mod = slot - row * jnp.int32(NEG_RATIO)
        tgt_lane = plsc.load_gather(tgt_v, [row])
        fb = tgt_lane + jnp.int32(1) + mod
        fb = jnp.where(fb >= jnp.int32(VOCAB), fb - jnp.int32(VOCAB), fb)
        acc = fb
        for k in reversed(range(ROUNDS)):
            off = k * NEG_W + sv * LANES
            j = jbuf[pl.ds(off, LANES)]
            wf = wfbuf[pl.ds(off, LANES)]
            u = ubuf[pl.ds(off, LANES)]
            u2 = u * u
            acc = jnp.where(u2 * u2 < wf * wf * wf,
                            jnp.where(j != tgt_lane, j, acc), acc)
        negidx[pl.ds(sv * LANES, LANES)] = acc

    cp_n0 = pltpu.async_copy(fc_hbm.at[negidx.at[pl.ds(0, 80)]],
                             negrows.at[pl.ds(0, 80)], sem)
    cp_n1 = pltpu.async_copy(fc_hbm.at[negidx.at[pl.ds(80, 80)]],
                             negrows.at[pl.ds(80, 80)], sem)
    cp_pos.wait()
    cp_n0.wait()
    cp_n1.wait()

    zero16 = iota.astype(jnp.float32) * jnp.float32(0.0)

    for sv in range(ROWS_W // LANES):
        sl = iota + jnp.int32(sv * LANES)

        def pos_step(h, acc, sl=sl):
            hv = iota * jnp.int32(0) + h
            fcv = plsc.load_gather(posrows, [sl, hv])
            ev = plsc.load_gather(emb_v, [sl, hv])
            return acc + fcv * ev

        poslog[pl.ds(sv * LANES, LANES)] = lax.fori_loop(
            0, EMBED_DIM, pos_step, zero16)

    for sv in range(NEG_W // LANES):
        sl = iota + jnp.int32(sv * LANES)
        row = (sl.astype(jnp.float32) * jnp.float32(1.0 / NEG_RATIO)
               ).astype(jnp.int32)

        def neg_step(h, acc, sl=sl, row=row):
            hv = iota * jnp.int32(0) + h
            fcv = plsc.load_gather(negrows, [sl, hv])
            ev = plsc.load_gather(emb_v, [row, hv])
            return acc + fcv * ev

        neglog[pl.ds(sv * LANES, LANES)] = lax.fori_loop(
            0, EMBED_DIM, neg_step, zero16)

    pltpu.sync_copy(poslog, posl_out.at[pl.ds(wid * ROWS_W, ROWS_W)])
    pltpu.sync_copy(neglog, negl_out.at[pl.ds(wid * NEG_W, NEG_W)])


def _make_sc_kernel():
    mesh = plsc.VectorSubcoreMesh(core_axis_name="c", subcore_axis_name="s")
    return pl.kernel(
        _sc_body,
        mesh=mesh,
        compiler_params=pltpu.CompilerParams(use_tc_tiling_on_sc=False,
                                             needs_layout_passes=False),
        out_type=[
            jax.ShapeDtypeStruct((BATCH,), jnp.float32),
            jax.ShapeDtypeStruct((BATCH * NEG_RATIO,), jnp.float32),
        ],
        scratch_types=[
            pltpu.VMEM((1, ROWS_W), jnp.int32),
            pltpu.VMEM((ROWS_W,), jnp.int32),
            pltpu.VMEM((ROWS_W, 2 * EMBED_DIM), jnp.float32),
            pltpu.VMEM((CAND_W,), jnp.int32),
            pltpu.VMEM((CAND_W,), jnp.float32),
            pltpu.VMEM((CAND_W,), jnp.float32),
            pltpu.VMEM((NEG_W,), jnp.int32),
            pltpu.VMEM((NEG_W, EMBED_DIM), jnp.float32),
            pltpu.VMEM((ROWS_W, EMBED_DIM), jnp.float32),
            pltpu.VMEM((ROWS_W,), jnp.float32),
            pltpu.VMEM((NEG_W,), jnp.float32),
            pltpu.SemaphoreType.DMA,
            pltpu.SemaphoreType.DMA,
        ],
    )


def _softplus(x):
    return jnp.maximum(x, 0.0) + jnp.log(1.0 + jnp.exp(-jnp.abs(x)))


def _tc_loss_body(posl_ref, negl_ref, out_ref):
    posl = posl_ref[...]
    negl = negl_ref[...]
    ploss = jnp.sum(_softplus(-posl))
    nloss = jnp.sum(_softplus(negl)) / jnp.float32(BATCH)
    out_ref[...] = (ploss + nloss).reshape(1, 1)


def kernel(embedding, target, word_freqs, fc):
    wf = word_freqs.astype(jnp.float32)

    embp, tgt2 = pl.pallas_call(
        _tc_pack_emb_body,
        out_shape=[
            jax.ShapeDtypeStruct((BATCH, 2 * EMBED_DIM), jnp.float32),
            jax.ShapeDtypeStruct((8, 128), jnp.int32),
        ],
    )(embedding, target.astype(jnp.int32))

    pos_logits, neg_logits = _make_sc_kernel()(embp, wf, tgt2, fc)

    loss = pl.pallas_call(
        _tc_loss_body,
        out_shape=jax.ShapeDtypeStruct((1, 1), jnp.float32),
    )(pos_logits.reshape(8, 128), neg_logits.reshape(40, 128))
    return loss[0, 0]


# speedup vs baseline: 1.0788x; 1.0788x over previous
import functools

import jax
import jax.numpy as jnp
from jax import lax
from jax.experimental import pallas as pl
from jax.experimental.pallas import tpu as pltpu
from jax.experimental.pallas import tpu_sc as plsc

VOCAB = 100000
EMBED_DIM = 64
NEG_RATIO = 5
BATCH = 1024

NUM_WORKERS = 32
ROWS_W = BATCH // NUM_WORKERS
NEG_W = ROWS_W * NEG_RATIO
ROUNDS = 8
CAND_W = ROUNDS * NEG_W
LANES = 16
CHUNK = 128

_INV24 = 1.0 / 16777216.0


def _fmix32(x):
    x = x ^ (x >> jnp.uint32(16))
    x = x * jnp.uint32(0x85EBCA6B)
    x = x ^ (x >> jnp.uint32(13))
    x = x * jnp.uint32(0xC2B2AE35)
    x = x ^ (x >> jnp.uint32(16))
    return x


def _u01(bits):
    return (bits >> jnp.uint32(8)).astype(jnp.int32).astype(jnp.float32) \
        * jnp.float32(_INV24)


def _tc_pack_emb_body(emb_ref, tgt_ref, out_ref, out_tgt_ref):
    e = emb_ref[...].reshape(BATCH, EMBED_DIM)
    out_ref[...] = jnp.concatenate([e, e], axis=1)
    out_tgt_ref[...] = tgt_ref[...].reshape(8, 128)


def _sc_body(emb_hbm, wf_hbm, tgt_hbm, fc_hbm, posl_out, negl_out,
             tgt2_v, tgt_v, emb_v, jbuf, ubuf, wfbuf, negidx, negrows, posrows,
             poslog, neglog, sem, sem2):
    wid = lax.axis_index("s") * 2 + lax.axis_index("c")
    iota = lax.iota(jnp.int32, LANES)

    pltpu.sync_copy(tgt_hbm.at[pl.ds(wid // 4, 1)], tgt2_v)
    zero16 = iota * jnp.int32(0)
    for k in range(ROWS_W // LANES):
        col = zero16 + (wid % 4) * ROWS_W + k * LANES + iota
        tgt_v[pl.ds(k * LANES, LANES)] = plsc.load_gather(tgt2_v, [zero16, col])
    pltpu.sync_copy(emb_hbm.at[pl.ds(wid * ROWS_W, ROWS_W)], emb_v)
    cp_pos = pltpu.async_copy(fc_hbm.at[tgt_v], posrows, sem2)

    base_u = (wid * CAND_W).astype(jnp.uint32)
    cps = []
    for ch in range(CAND_W // CHUNK):
        for cv in range(CHUNK // LANES):
            off = ch * CHUNK + cv * LANES
            g = (iota + off).astype(jnp.uint32) + base_u
            jbits = _fmix32(g * jnp.uint32(0x9E3779B1))
            ubits = _fmix32((g ^ jnp.uint32(0xDEADBEEF)) * jnp.uint32(0x85EBCA77))
            j = (_u01(jbits) * jnp.float32(VOCAB)).astype(jnp.int32)
            jbuf[pl.ds(off, LANES)] = j
            ubuf[pl.ds(off, LANES)] = _u01(ubits)
        cps.append(pltpu.async_copy(
            wf_hbm.at[jbuf.at[pl.ds(ch * CHUNK, CHUNK)]],
            wfbuf.at[pl.ds(ch * CHUNK, CHUNK)], sem))
    for cp in cps:
        cp.wait()

    for sv in range(NEG_W // LANES):
        slot = iota + jnp.int32(sv * LANES)
        row = (slot.astype(jnp.float32) * jnp.float32(1.0 / NEG_RATIO)
               ).astype(jnp.int32)
        mod = slot - row * jnp.int32(NEG_RATIO)
        tgt_lane = plsc.load_gather(tgt_v, [row])
        fb = tgt_lane + jnp.int32(1) + mod
        fb = jnp.where(fb >= jnp.int32(VOCAB), fb - jnp.int32(VOCAB), fb)
        acc = fb
        for k in reversed(range(ROUNDS)):
            off = k * NEG_W + sv * LANES
            j = jbuf[pl.ds(off, LANES)]
            wf = wfbuf[pl.ds(off, LANES)]
            u = ubuf[pl.ds(off, LANES)]
            u2 = u * u
            acc = jnp.where(u2 * u2 < wf * wf * wf,
                            jnp.where(j != tgt_lane, j, acc), acc)
        negidx[pl.ds(sv * LANES, LANES)] = acc

    cp_n0 = pltpu.async_copy(fc_hbm.at[negidx.at[pl.ds(0, 80)]],
                             negrows.at[pl.ds(0, 80)], sem)
    cp_n1 = pltpu.async_copy(fc_hbm.at[negidx.at[pl.ds(80, 80)]],
                             negrows.at[pl.ds(80, 80)], sem)
    cp_pos.wait()
    cp_n0.wait()
    cp_n1.wait()

    zerof = iota.astype(jnp.float32) * jnp.float32(0.0)

    for sv in range(ROWS_W // LANES):
        sl = iota + jnp.int32(sv * LANES)

        def pos_step(h, acc, sl=sl):
            hv = iota * jnp.int32(0) + h
            fcv = plsc.load_gather(posrows, [sl, hv])
            ev = plsc.load_gather(emb_v, [sl, hv])
            return acc + fcv * ev

        poslog[pl.ds(sv * LANES, LANES)] = lax.fori_loop(
            0, EMBED_DIM, pos_step, zerof)

    for sv in range(NEG_W // LANES):
        sl = iota + jnp.int32(sv * LANES)
        row = (sl.astype(jnp.float32) * jnp.float32(1.0 / NEG_RATIO)
               ).astype(jnp.int32)

        def neg_step(h, acc, sl=sl, row=row):
            hv = iota * jnp.int32(0) + h
            fcv = plsc.load_gather(negrows, [sl, hv])
            ev = plsc.load_gather(emb_v, [row, hv])
            return acc + fcv * ev

        neglog[pl.ds(sv * LANES, LANES)] = lax.fori_loop(
            0, EMBED_DIM, neg_step, zerof)

    pltpu.sync_copy(poslog, posl_out.at[pl.ds(wid * ROWS_W, ROWS_W)])
    pltpu.sync_copy(neglog, negl_out.at[pl.ds(wid * NEG_W, NEG_W)])


def _make_sc_kernel():
    mesh = plsc.VectorSubcoreMesh(core_axis_name="c", subcore_axis_name="s")
    return pl.kernel(
        _sc_body,
        mesh=mesh,
        compiler_params=pltpu.CompilerParams(use_tc_tiling_on_sc=True,
                                             needs_layout_passes=False),
        out_type=[
            jax.ShapeDtypeStruct((BATCH,), jnp.float32),
            jax.ShapeDtypeStruct((BATCH * NEG_RATIO,), jnp.float32),
        ],
        scratch_types=[
            pltpu.VMEM((1, 128), jnp.int32),
            pltpu.VMEM((ROWS_W,), jnp.int32),
            pltpu.VMEM((ROWS_W, 2 * EMBED_DIM), jnp.float32),
            pltpu.VMEM((CAND_W,), jnp.int32),
            pltpu.VMEM((CAND_W,), jnp.float32),
            pltpu.VMEM((CAND_W,), jnp.float32),
            pltpu.VMEM((NEG_W,), jnp.int32),
            pltpu.VMEM((NEG_W, 2 * EMBED_DIM), jnp.float32),
            pltpu.VMEM((ROWS_W, 2 * EMBED_DIM), jnp.float32),
            pltpu.VMEM((ROWS_W,), jnp.float32),
            pltpu.VMEM((NEG_W,), jnp.float32),
            pltpu.SemaphoreType.DMA,
            pltpu.SemaphoreType.DMA,
        ],
    )


def _softplus(x):
    return jnp.maximum(x, 0.0) + jnp.log(1.0 + jnp.exp(-jnp.abs(x)))


def _tc_loss_body(posl_ref, negl_ref, out_ref):
    posl = posl_ref[...]
    negl = negl_ref[...]
    ploss = jnp.sum(_softplus(-posl))
    nloss = jnp.sum(_softplus(negl)) / jnp.float32(BATCH)
    out_ref[...] = (ploss + nloss).reshape(1, 1)


def kernel(embedding, target, word_freqs, fc):
    wf = word_freqs.astype(jnp.float32)

    embp, tgt2 = pl.pallas_call(
        _tc_pack_emb_body,
        out_shape=[
            jax.ShapeDtypeStruct((BATCH, 2 * EMBED_DIM), jnp.float32),
            jax.ShapeDtypeStruct((8, 128), jnp.int32),
        ],
    )(embedding, target.astype(jnp.int32))

    fcp = jnp.pad(fc, ((0, 0), (0, EMBED_DIM)))
    pos_logits, neg_logits = _make_sc_kernel()(embp, wf, tgt2, fcp)

    loss = pl.pallas_call(
        _tc_loss_body,
        out_shape=jax.ShapeDtypeStruct((1, 1), jnp.float32),
    )(pos_logits.reshape(8, 128), neg_logits.reshape(40, 128))
    return loss[0, 0]
